# baseline (speedup 1.0000x reference)
import functools

import jax
import jax.numpy as jnp
from jax import lax
from jax.experimental import pallas as pl
from jax.experimental.pallas import tpu as pltpu
from jax.experimental.pallas import tpu_sc as plsc

NC = 2
NS = 16
L = 16
NW = NC * NS


def _mm_body(h2, x_ref, w_ref, o1_ref, o2_ref):
    xw = jnp.dot(x_ref[...], w_ref[...], preferred_element_type=jnp.float32)
    o1_ref[...] = xw[:, :h2]
    o2_ref[...] = xw[:, h2:]


def _colsum(p):
    ones = jnp.ones((p.shape[0], 1), jnp.float32)
    return lax.dot_general(p, ones, (((0,), (0,)), ((), ())),
                           preferred_element_type=jnp.float32)


def _prep_body(degp_ref, xw1_ref, dinv_ref, dinvr_ref, u_ref):
    deg = _colsum(degp_ref[...])
    dinv = lax.rsqrt(deg)
    dinv_ref[...] = dinv
    dinvr_ref[...] = dinv.T
    u_ref[...] = (xw1_ref[...] * dinv).astype(jnp.bfloat16)


def _mid_body(npad, acc_ref, xf1_ref, dinv_ref, dinvr_ref, b1_ref, bf1_ref,
              w2a_ref, w2b_ref, bb_ref, v_ref, pre_ref):
    dinv = dinv_ref[...]
    h1 = jnp.maximum(dinv * (acc_ref[0] + acc_ref[1]).astype(jnp.float32)
                     + b1_ref[...], 0.0)
    hf = jnp.maximum(xf1_ref[...] + bf1_ref[...], 0.0)
    sf = (jnp.dot(h1, w2a_ref[...], preferred_element_type=jnp.float32)
          + jnp.dot(hf, w2b_ref[...], preferred_element_type=jnp.float32))
    sft = sf.T
    v_ref[...] = (dinvr_ref[...] * sft[0:1]).reshape(npad)
    pre_ref[...] = sft[1:2] + bb_ref[...]


def _final_body(n, npad, p2_ref, dinvr_ref, pre_ref, o_ref):
    ones = jnp.ones((1, p2_ref.shape[0]), jnp.float32)
    tot = jnp.dot(ones, p2_ref[...], preferred_element_type=jnp.float32)
    out = dinvr_ref[...] * tot + pre_ref[...]
    o_ref[...] = out[:, :n].reshape(n)


def _sc_scalar_body(cpt, npad, ei_hbm, v_hbm, out_hbm,
                    src_v, dst_v, v_v, acc_v):
    c = lax.axis_index("c")
    s = lax.axis_index("s")
    wid = s * NC + c
    pltpu.sync_copy(ei_hbm.at[0, wid], src_v)
    pltpu.sync_copy(ei_hbm.at[1, wid], dst_v)
    pltpu.sync_copy(v_hbm, v_v)

    z16 = jnp.zeros((L,), jnp.float32)

    def zero_body(i, carry):
        for k in range(8):
            acc_v[pl.ds((i * 8 + k) * L, L)] = z16
        return carry

    lax.fori_loop(0, npad // (L * 8), zero_body, 0)

    def edge_body(r, carry):
        for k in range(128 // L):
            sidx = src_v[r, pl.ds(k * L, L)]
            didx = dst_v[r, pl.ds(k * L, L)]
            vals = plsc.load_gather(v_v, [sidx])
            plsc.addupdate_scatter(acc_v, [didx], vals)
        return carry

    lax.fori_loop(0, cpt, edge_body, 0)
    pltpu.sync_copy(acc_v, out_hbm.at[wid])


def _sc_deg_body(cpt, npad, ei_hbm, out_hbm, dst_v, acc_v):
    c = lax.axis_index("c")
    s = lax.axis_index("s")
    wid = s * NC + c
    pltpu.sync_copy(ei_hbm.at[1, wid], dst_v)

    z16 = jnp.zeros((L,), jnp.float32)
    one16 = jnp.ones((L,), jnp.float32)

    def zero_body(i, carry):
        for k in range(8):
            acc_v[pl.ds((i * 8 + k) * L, L)] = z16
        return carry

    lax.fori_loop(0, npad // (L * 8), zero_body, 0)

    def edge_body(r, carry):
        for k in range(128 // L):
            didx = dst_v[r, pl.ds(k * L, L)]
            plsc.addupdate_scatter(acc_v, [didx], one16)
        return carry

    lax.fori_loop(0, cpt, edge_body, 0)
    pltpu.sync_copy(acc_v, out_hbm.at[wid])


def _sc_rows_body(cpt, npad, h2, ei_hbm, u_hbm, out_hbm,
                  src_v, dst_v, rows_v, acc_sh, sem0, sem1):
    c = lax.axis_index("c")
    s = lax.axis_index("s")
    wid = s * NC + c
    rpt = npad // NS

    z32 = jnp.zeros((2 * L,), jnp.bfloat16)

    def zero_body(r, carry):
        for k in range(h2 // (2 * L)):
            rows_v[0, r, pl.ds(k * 2 * L, 2 * L)] = z32
        return carry

    lax.fori_loop(0, 128, zero_body, 0)
    for q in range(rpt // 128):
        pltpu.sync_copy(rows_v.at[0], acc_sh.at[pl.ds(s * rpt + q * 128, 128)])
    plsc.subcore_barrier()

    pltpu.sync_copy(ei_hbm.at[0, wid], src_v)
    pltpu.sync_copy(ei_hbm.at[1, wid], dst_v)

    pltpu.async_copy(u_hbm.at[src_v.at[0]], rows_v.at[0], sem0)

    def chunk_body(j2, carry):
        j = 2 * j2
        pltpu.async_copy(u_hbm.at[src_v.at[j + 1]], rows_v.at[1], sem1)
        pltpu.make_async_copy(u_hbm.at[src_v.at[j]], rows_v.at[0], sem0).wait()
        pltpu.sync_copy(rows_v.at[0], acc_sh.at[dst_v.at[j]], add=True)

        @pl.when(j2 < cpt // 2 - 1)
        def _():
            pltpu.async_copy(u_hbm.at[src_v.at[j + 2]], rows_v.at[0], sem0)

        pltpu.make_async_copy(u_hbm.at[src_v.at[j + 1]], rows_v.at[1],
                              sem1).wait()
        pltpu.sync_copy(rows_v.at[1], acc_sh.at[dst_v.at[j + 1]], add=True)
        return carry

    lax.fori_loop(0, cpt // 2, chunk_body, 0)
    plsc.subcore_barrier()
    pltpu.sync_copy(acc_sh.at[pl.ds(s * rpt, rpt)],
                    out_hbm.at[c, pl.ds(s * rpt, rpt)])


def kernel(x, edge_index, W1, b1, Wf1, bf1, W2, b2, Wf2, bf2):
    n, d = x.shape
    e = edge_index.shape[1]
    h2 = W1.shape[1]
    h = 2 * h2

    npad = ((n + 255) // 256) * 256
    ne = e + npad
    ept = ((ne // NW + 255) // 256) * 256
    epad = ept * NW
    cpt = ept // 128

    self_idx = jnp.arange(npad, dtype=jnp.int32)
    pad_idx = n + (jnp.arange(epad - ne, dtype=jnp.int32) % 128)
    ei_pad = jnp.concatenate(
        [edge_index, jnp.broadcast_to(self_idx, (2, npad)),
         jnp.broadcast_to(pad_idx, (2, epad - ne))], axis=1)
    ei4 = ei_pad.reshape(2, NW, -1, 128)
    x_pad = jnp.pad(x, ((0, npad - n), (0, 0)))
    wc = jnp.concatenate([W1, Wf1], axis=1)
    w2c = jnp.concatenate([W2, Wf2], axis=1)

    f32 = jnp.float32
    bf16 = jnp.bfloat16
    mesh = plsc.VectorSubcoreMesh(core_axis_name="c", subcore_axis_name="s")

    sc_scalar = pl.kernel(
        functools.partial(_sc_scalar_body, cpt, npad),
        out_type=jax.ShapeDtypeStruct((NW, npad), f32),
        mesh=mesh,
        compiler_params=pltpu.CompilerParams(needs_layout_passes=False),
        scratch_types=[
            pltpu.VMEM((cpt, 128), jnp.int32),
            pltpu.VMEM((cpt, 128), jnp.int32),
            pltpu.VMEM((npad,), f32),
            pltpu.VMEM((npad,), f32),
        ],
    )

    sc_deg = pl.kernel(
        functools.partial(_sc_deg_body, cpt, npad),
        out_type=jax.ShapeDtypeStruct((NW, npad), f32),
        mesh=mesh,
        compiler_params=pltpu.CompilerParams(needs_layout_passes=False),
        scratch_types=[
            pltpu.VMEM((cpt, 128), jnp.int32),
            pltpu.VMEM((npad,), f32),
        ],
    )

    sc_rows = pl.kernel(
        functools.partial(_sc_rows_body, cpt, npad, h2),
        out_type=jax.ShapeDtypeStruct((NC, npad, h2), bf16),
        mesh=mesh,
        compiler_params=pltpu.CompilerParams(needs_layout_passes=False,
                                             use_tc_tiling_on_sc=False),
        scratch_types=[
            pltpu.VMEM((cpt, 128), jnp.int32),
            pltpu.VMEM((cpt, 128), jnp.int32),
            pltpu.VMEM((2, 128, h2), bf16),
            pltpu.VMEM_SHARED((npad, h2), bf16),
            pltpu.SemaphoreType.DMA,
            pltpu.SemaphoreType.DMA,
        ],
    )

    xw1, xf1 = pl.pallas_call(
        functools.partial(_mm_body, h2),
        out_shape=(
            jax.ShapeDtypeStruct((npad, h2), f32),
            jax.ShapeDtypeStruct((npad, h2), f32),
        ),
    )(x_pad, wc)
    degp = sc_deg(ei4)

    dinv, dinvr, u = pl.pallas_call(
        _prep_body,
        out_shape=(
            jax.ShapeDtypeStruct((npad, 1), f32),
            jax.ShapeDtypeStruct((1, npad), f32),
            jax.ShapeDtypeStruct((npad, h2), bf16),
        ),
    )(degp, xw1)

    acc = sc_rows(ei4, u)

    v, pre = pl.pallas_call(
        functools.partial(_mid_body, npad),
        out_shape=(
            jax.ShapeDtypeStruct((npad,), f32),
            jax.ShapeDtypeStruct((1, npad), f32),
        ),
    )(acc, xf1, dinv, dinvr, b1.reshape(1, h2), bf1.reshape(1, h2),
      w2c[:h2], w2c[h2:], (b2 + bf2).reshape(1, 1))

    p2 = sc_scalar(ei4, v)

    out = pl.pallas_call(
        functools.partial(_final_body, n, npad),
        out_shape=jax.ShapeDtypeStruct((n,), f32),
    )(p2, dinvr, pre)
    return out


# speedup vs baseline: 1.1476x; 1.1476x over previous
import functools

import jax
import jax.numpy as jnp
import numpy as np
from jax import lax
from jax.experimental import pallas as pl
from jax.experimental.pallas import tpu as pltpu
from jax.experimental.pallas import tpu_sc as plsc

NC = 2
NS = 16
L = 16
NW = NC * NS


def _mm_body(h2, x_ref, w_ref, o1_ref, o2_ref):
    xw = jnp.dot(x_ref[...], w_ref[...], preferred_element_type=jnp.float32)
    o1_ref[...] = xw[:, :h2]
    o2_ref[...] = xw[:, h2:]


def _colsum(p):
    ones = jnp.ones((p.shape[0], 1), jnp.float32)
    return lax.dot_general(p, ones, (((0,), (0,)), ((), ())),
                           preferred_element_type=jnp.float32)


def _prep_body(degp_ref, xw1_ref, dinv_ref, dinvr_ref, u_ref):
    deg = _colsum(degp_ref[...])
    dinv = lax.rsqrt(deg)
    dinv_ref[...] = dinv
    dinvr_ref[...] = dinv.T
    u_ref[...] = (xw1_ref[...] * dinv).astype(jnp.bfloat16)


def _mid_body(npad, acc_ref, xf1_ref, dinv_ref, dinvr_ref, b1_ref, bf1_ref,
              w2a_ref, w2b_ref, bb_ref, v_ref, pre_ref):
    dinv = dinv_ref[...]
    h1 = jnp.maximum(dinv * (acc_ref[0] + acc_ref[1]).astype(jnp.float32)
                     + b1_ref[...], 0.0)
    hf = jnp.maximum(xf1_ref[...] + bf1_ref[...], 0.0)
    sf = (jnp.dot(h1, w2a_ref[...], preferred_element_type=jnp.float32)
          + jnp.dot(hf, w2b_ref[...], preferred_element_type=jnp.float32))
    sft = sf.T
    v_ref[...] = (dinvr_ref[...] * sft[0:1]).reshape(npad)
    pre_ref[...] = sft[1:2] + bb_ref[...]


def _final_body(n, npad, p2_ref, dinvr_ref, pre_ref, o_ref):
    ones = jnp.ones((1, p2_ref.shape[0]), jnp.float32)
    tot = jnp.dot(ones, p2_ref[...], preferred_element_type=jnp.float32)
    out = dinvr_ref[...] * tot + pre_ref[...]
    o_ref[...] = out[:, :n].reshape(n)


def _sc_scalar_body(cpt, npad, ei_hbm, v_hbm, out_hbm,
                    src_v, dst_v, v_v, acc_v):
    c = lax.axis_index("c")
    s = lax.axis_index("s")
    wid = s * NC + c
    pltpu.sync_copy(ei_hbm.at[0, wid], src_v)
    pltpu.sync_copy(ei_hbm.at[1, wid], dst_v)
    pltpu.sync_copy(v_hbm, v_v)

    z16 = jnp.zeros((L,), jnp.float32)

    def zero_body(i, carry):
        for k in range(8):
            acc_v[pl.ds((i * 8 + k) * L, L)] = z16
        return carry

    lax.fori_loop(0, npad // (L * 8), zero_body, 0)

    def edge_body(r, carry):
        for k in range(128 // L):
            sidx = src_v[r, pl.ds(k * L, L)]
            didx = dst_v[r, pl.ds(k * L, L)]
            vals = plsc.load_gather(v_v, [sidx])
            plsc.addupdate_scatter(acc_v, [didx], vals)
        return carry

    lax.fori_loop(0, cpt, edge_body, 0)
    pltpu.sync_copy(acc_v, out_hbm.at[wid])


def _sc_deg_body(cpt, npad, ei_hbm, out_hbm, dst_v, acc_v):
    c = lax.axis_index("c")
    s = lax.axis_index("s")
    wid = s * NC + c
    pltpu.sync_copy(ei_hbm.at[1, wid], dst_v)

    z16 = jnp.zeros((L,), jnp.float32)
    one16 = jnp.ones((L,), jnp.float32)

    def zero_body(i, carry):
        for k in range(8):
            acc_v[pl.ds((i * 8 + k) * L, L)] = z16
        return carry

    lax.fori_loop(0, npad // (L * 8), zero_body, 0)

    def edge_body(r, carry):
        for k in range(128 // L):
            didx = dst_v[r, pl.ds(k * L, L)]
            plsc.addupdate_scatter(acc_v, [didx], one16)
        return carry

    lax.fori_loop(0, cpt, edge_body, 0)
    pltpu.sync_copy(acc_v, out_hbm.at[wid])


def _sc_rows_body(cpt, npad, h2, ei_hbm, u_hbm, out_hbm,
                  src_v, dst_v, rows_v, acc_sh, sem0, sem1):
    c = lax.axis_index("c")
    s = lax.axis_index("s")
    wid = s * NC + c
    rpt = npad // NS

    z32 = jnp.zeros((2 * L,), jnp.bfloat16)

    def zero_body(r, carry):
        for k in range(h2 // (2 * L)):
            rows_v[0, r, pl.ds(k * 2 * L, 2 * L)] = z32
        return carry

    lax.fori_loop(0, 128, zero_body, 0)
    for q in range(rpt // 128):
        pltpu.sync_copy(rows_v.at[0], acc_sh.at[pl.ds(s * rpt + q * 128, 128)])
    plsc.subcore_barrier()

    pltpu.sync_copy(ei_hbm.at[0, wid], src_v)
    pltpu.sync_copy(ei_hbm.at[1, wid], dst_v)

    pltpu.async_copy(u_hbm.at[src_v.at[0]], rows_v.at[0], sem0)

    def chunk_body(j2, carry):
        j = 2 * j2
        pltpu.async_copy(u_hbm.at[src_v.at[j + 1]], rows_v.at[1], sem1)
        pltpu.make_async_copy(u_hbm.at[src_v.at[j]], rows_v.at[0], sem0).wait()
        pltpu.sync_copy(rows_v.at[0], acc_sh.at[dst_v.at[j]], add=True)

        @pl.when(j2 < cpt // 2 - 1)
        def _():
            pltpu.async_copy(u_hbm.at[src_v.at[j + 2]], rows_v.at[0], sem0)

        pltpu.make_async_copy(u_hbm.at[src_v.at[j + 1]], rows_v.at[1],
                              sem1).wait()
        pltpu.sync_copy(rows_v.at[1], acc_sh.at[dst_v.at[j + 1]], add=True)
        return carry

    lax.fori_loop(0, cpt // 2, chunk_body, 0)
    plsc.subcore_barrier()
    pltpu.sync_copy(acc_sh.at[pl.ds(s * rpt, rpt)],
                    out_hbm.at[c, pl.ds(s * rpt, rpt)])


def kernel(x, edge_index, W1, b1, Wf1, bf1, W2, b2, Wf2, bf2):
    n, d = x.shape
    e = edge_index.shape[1]
    h2 = W1.shape[1]
    h = 2 * h2

    npad = ((n + 255) // 256) * 256
    ne = e + npad
    ept = ((ne // NW + 255) // 256) * 256
    epad = ept * NW
    cpt = ept // 128

    tail = np.concatenate(
        [np.arange(npad, dtype=np.int32),
         n + (np.arange(epad - ne, dtype=np.int32) % 128)])
    ei_pad = jnp.concatenate(
        [edge_index, jnp.asarray(np.tile(tail, (2, 1)))], axis=1)
    ei4 = ei_pad.reshape(2, NW, -1, 128)
    x_pad = jnp.pad(x, ((0, npad - n), (0, 0)))
    wc = jnp.concatenate([W1, Wf1], axis=1)
    w2c = jnp.concatenate([W2, Wf2], axis=1)

    f32 = jnp.float32
    bf16 = jnp.bfloat16
    mesh = plsc.VectorSubcoreMesh(core_axis_name="c", subcore_axis_name="s")

    sc_scalar = pl.kernel(
        functools.partial(_sc_scalar_body, cpt, npad),
        out_type=jax.ShapeDtypeStruct((NW, npad), f32),
        mesh=mesh,
        compiler_params=pltpu.CompilerParams(needs_layout_passes=False),
        scratch_types=[
            pltpu.VMEM((cpt, 128), jnp.int32),
            pltpu.VMEM((cpt, 128), jnp.int32),
            pltpu.VMEM((npad,), f32),
            pltpu.VMEM((npad,), f32),
        ],
    )

    sc_deg = pl.kernel(
        functools.partial(_sc_deg_body, cpt, npad),
        out_type=jax.ShapeDtypeStruct((NW, npad), f32),
        mesh=mesh,
        compiler_params=pltpu.CompilerParams(needs_layout_passes=False),
        scratch_types=[
            pltpu.VMEM((cpt, 128), jnp.int32),
            pltpu.VMEM((npad,), f32),
        ],
    )

    sc_rows = pl.kernel(
        functools.partial(_sc_rows_body, cpt, npad, h2),
        out_type=jax.ShapeDtypeStruct((NC, npad, h2), bf16),
        mesh=mesh,
        compiler_params=pltpu.CompilerParams(needs_layout_passes=False,
                                             use_tc_tiling_on_sc=False),
        scratch_types=[
            pltpu.VMEM((cpt, 128), jnp.int32),
            pltpu.VMEM((cpt, 128), jnp.int32),
            pltpu.VMEM((2, 128, h2), bf16),
            pltpu.VMEM_SHARED((npad, h2), bf16),
            pltpu.SemaphoreType.DMA,
            pltpu.SemaphoreType.DMA,
        ],
    )

    xw1, xf1 = pl.pallas_call(
        functools.partial(_mm_body, h2),
        out_shape=(
            jax.ShapeDtypeStruct((npad, h2), f32),
            jax.ShapeDtypeStruct((npad, h2), f32),
        ),
    )(x_pad, wc)
    degp = sc_deg(ei4)

    dinv, dinvr, u = pl.pallas_call(
        _prep_body,
        out_shape=(
            jax.ShapeDtypeStruct((npad, 1), f32),
            jax.ShapeDtypeStruct((1, npad), f32),
            jax.ShapeDtypeStruct((npad, h2), bf16),
        ),
    )(degp, xw1)

    acc = sc_rows(ei4, u)

    v, pre = pl.pallas_call(
        functools.partial(_mid_body, npad),
        out_shape=(
            jax.ShapeDtypeStruct((npad,), f32),
            jax.ShapeDtypeStruct((1, npad), f32),
        ),
    )(acc, xf1, dinv, dinvr, b1.reshape(1, h2), bf1.reshape(1, h2),
      w2c[:h2], w2c[h2:], (b2 + bf2).reshape(1, 1))

    p2 = sc_scalar(ei4, v)

    out = pl.pallas_call(
        functools.partial(_final_body, n, npad),
        out_shape=jax.ShapeDtypeStruct((n,), f32),
    )(p2, dinvr, pre)
    return out
